# Initial kernel scaffold; baseline (speedup 1.0000x reference)
import jax
import jax.numpy as jnp
from jax.experimental import pallas as pl


def kernel(x, edge_index, W1, b1, gamma, beta, W2, b2):
    raise NotImplementedError("write your pallas kernel here")


# speedup vs baseline: 7.4374x; 7.4374x over previous
import jax
import jax.numpy as jnp
from jax import lax
from jax.experimental import pallas as pl
from jax.experimental.pallas import tpu as pltpu
from jax.experimental.pallas import tpu_sc as plsc

N = 10000
NP = 10240
E = 320000
NT = 32
CH = 128
NCH = 79
EPT = NCH * CH
EP = EPT * NT
DEG_NCH = 157
EPD = DEG_NCH * CH * 16
D_IN = 128
D_OUT = 512
RB = 1280
F32 = jnp.float32


def _mesh():
    return plsc.VectorSubcoreMesh(core_axis_name="c", subcore_axis_name="s")


def _deg_body(cidx_hbm, deg_hbm, acc, cidx_v, onesb):
    c = lax.axis_index("c")
    s = lax.axis_index("s")
    zero16 = jnp.zeros((16,), F32)
    one16 = jnp.full((16,), 1.0, F32)

    def zrow(i, _):
        for jj in range(8):
            onesb[i, pl.ds(jj * 16, 16)] = zero16
        return 0

    lax.fori_loop(0, 128, zrow, 0)

    def zero_acc(k, _):
        pltpu.sync_copy(onesb, acc.at[pl.ds(s * 640 + k * 128, 128)])
        return 0

    lax.fori_loop(0, 5, zero_acc, 0)

    def orow(i, _):
        for jj in range(8):
            onesb[i, pl.ds(jj * 16, 16)] = one16
        return 0

    lax.fori_loop(0, 128, orow, 0)
    plsc.subcore_barrier()

    pltpu.sync_copy(cidx_hbm.at[c, s], cidx_v)

    def add_chunk(j, _):
        pltpu.sync_copy(onesb, acc.at[cidx_v.at[j]], add=True)
        return 0

    lax.fori_loop(0, DEG_NCH, add_chunk, 0)
    plsc.subcore_barrier()

    pltpu.sync_copy(acc.at[pl.ds(s * 640, 640)],
                    deg_hbm.at[c, pl.ds(s * 640, 640)])


def _deg_call(cidx):
    fn = pl.kernel(
        _deg_body,
        out_type=jax.ShapeDtypeStruct((2, NP, 128), F32),
        mesh=_mesh(),
        scratch_types=[
            pltpu.VMEM_SHARED((NP, 128), F32),
            pltpu.VMEM((DEG_NCH, 128), jnp.int32),
            pltpu.VMEM((128, 128), F32),
        ],
    )
    return fn(cidx)


def _spmv_body(hs_hbm, srcr_hbm, dstr_hbm, y_hbm, acc, srcv, dstv, rowb, sem):
    c = lax.axis_index("c")
    s = lax.axis_index("s")
    wid = c * 16 + s

    zero16 = jnp.zeros((16,), F32)

    def init_row(i, _):
        for jj in range(8):
            rowb[i, pl.ds(jj * 16, 16)] = zero16
        return 0

    lax.fori_loop(0, 128, init_row, 0)

    def zero_acc(k, _):
        pltpu.sync_copy(rowb, acc.at[pl.ds(s * 640 + k * 128, 128)])
        return 0

    lax.fori_loop(0, 5, zero_acc, 0)
    plsc.subcore_barrier()

    pltpu.sync_copy(srcr_hbm.at[wid], srcv)
    pltpu.sync_copy(dstr_hbm.at[wid], dstv)

    def edge_chunk(j, _):
        pltpu.async_copy(hs_hbm.at[srcv.at[j]], rowb, sem).wait()
        pltpu.sync_copy(rowb, acc.at[dstv.at[j]], add=True)
        return 0

    lax.fori_loop(0, NCH, edge_chunk, 0)
    plsc.subcore_barrier()

    pltpu.sync_copy(acc.at[pl.ds(s * 640, 640)],
                    y_hbm.at[c, pl.ds(s * 640, 640)])


def _spmv_call(hs, srcr, dstr):
    fn = pl.kernel(
        _spmv_body,
        out_type=jax.ShapeDtypeStruct((2, NP, D_IN), F32),
        mesh=_mesh(),
        scratch_types=[
            pltpu.VMEM_SHARED((NP, D_IN), F32),
            pltpu.VMEM((NCH, CH), jnp.int32),
            pltpu.VMEM((NCH, CH), jnp.int32),
            pltpu.VMEM((CH, D_IN), F32),
            pltpu.SemaphoreType.DMA,
        ],
    )
    return fn(hs, srcr, dstr)


def _scale_in_body(dsrc, ddst, x, hs, bmat):
    a = lax.rsqrt(jnp.maximum(dsrc[...], 1.0))
    hs[...] = x[...] * a
    bmat[...] = lax.rsqrt(jnp.maximum(ddst[...], 1.0))


def _scale_in(dsrc, ddst, xp):
    grid = NP // RB
    return pl.pallas_call(
        _scale_in_body,
        grid=(grid,),
        in_specs=[
            pl.BlockSpec((RB, D_IN), lambda r: (r, 0)),
            pl.BlockSpec((RB, D_IN), lambda r: (r, 0)),
            pl.BlockSpec((RB, D_IN), lambda r: (r, 0)),
        ],
        out_specs=[
            pl.BlockSpec((RB, D_IN), lambda r: (r, 0)),
            pl.BlockSpec((RB, D_IN), lambda r: (r, 0)),
        ],
        out_shape=[
            jax.ShapeDtypeStruct((NP, D_IN), F32),
            jax.ShapeDtypeStruct((NP, D_IN), F32),
        ],
    )(dsrc, ddst, xp)


def _combine_body(y, bmat, x, dsrc, hs2):
    h1 = (y[0] + y[1]) * bmat[...] + x[...]
    a = lax.rsqrt(jnp.maximum(dsrc[...], 1.0))
    hs2[...] = h1 * a


def _combine(y1, bmat, xp, dsrc):
    grid = NP // RB
    return pl.pallas_call(
        _combine_body,
        grid=(grid,),
        in_specs=[
            pl.BlockSpec((2, RB, D_IN), lambda r: (0, r, 0)),
            pl.BlockSpec((RB, D_IN), lambda r: (r, 0)),
            pl.BlockSpec((RB, D_IN), lambda r: (r, 0)),
            pl.BlockSpec((RB, D_IN), lambda r: (r, 0)),
        ],
        out_specs=pl.BlockSpec((RB, D_IN), lambda r: (r, 0)),
        out_shape=jax.ShapeDtypeStruct((NP, D_IN), F32),
    )(y1, bmat, xp, dsrc)


def _mlp_body(y, bmat, x, w1, b1, g, bt, w2, b2, out):
    h2 = (y[0] + y[1]) * bmat[...] + x[...]
    e = jnp.dot(h2, w1[...], preferred_element_type=F32,
                precision=lax.Precision.HIGHEST) + b1[...]
    e = jnp.maximum(e, 0.0)
    mu = jnp.mean(e, axis=-1, keepdims=True)
    var = jnp.mean((e - mu) ** 2, axis=-1, keepdims=True)
    e = g[...] * (e - mu) * lax.rsqrt(var + 1e-5) + bt[...]
    o = jnp.dot(e, w2[...], preferred_element_type=F32,
                precision=lax.Precision.HIGHEST) + b2[...]
    nrm = jnp.sqrt(jnp.sum(o * o, axis=-1, keepdims=True))
    out[...] = o / jnp.maximum(nrm, 1e-8)


def _mlp(y2, bmat, xp, W1, b1, gamma, beta, W2, b2):
    grid = NP // RB
    return pl.pallas_call(
        _mlp_body,
        grid=(grid,),
        in_specs=[
            pl.BlockSpec((2, RB, D_IN), lambda r: (0, r, 0)),
            pl.BlockSpec((RB, D_IN), lambda r: (r, 0)),
            pl.BlockSpec((RB, D_IN), lambda r: (r, 0)),
            pl.BlockSpec((D_IN, D_OUT), lambda r: (0, 0)),
            pl.BlockSpec((1, D_OUT), lambda r: (0, 0)),
            pl.BlockSpec((1, D_OUT), lambda r: (0, 0)),
            pl.BlockSpec((1, D_OUT), lambda r: (0, 0)),
            pl.BlockSpec((D_OUT, D_OUT), lambda r: (0, 0)),
            pl.BlockSpec((1, D_OUT), lambda r: (0, 0)),
        ],
        out_specs=pl.BlockSpec((RB, D_OUT), lambda r: (r, 0)),
        out_shape=jax.ShapeDtypeStruct((NP, D_OUT), F32),
    )(y2, bmat, xp, W1, b1, gamma, beta, W2, b2)


def kernel(x, edge_index, W1, b1, gamma, beta, W2, b2):
    src = edge_index[0]
    dst = edge_index[1]
    pad_spmv = jnp.full((EP - E,), N, dtype=jnp.int32)
    src_p = jnp.concatenate([src, pad_spmv])
    dst_p = jnp.concatenate([dst, pad_spmv])
    srcr = src_p.reshape(NT, NCH, CH)
    dstr = dst_p.reshape(NT, NCH, CH)
    pad_deg = jnp.full((EPD - E,), N, dtype=jnp.int32)
    cidx = jnp.stack([
        jnp.concatenate([src, pad_deg]),
        jnp.concatenate([dst, pad_deg]),
    ]).reshape(2, 16, DEG_NCH, CH)
    xp = jnp.pad(x, ((0, NP - N), (0, 0)))

    degr = _deg_call(cidx)
    hs1, bmat = _scale_in(degr[0], degr[1], xp)
    y1 = _spmv_call(hs1, srcr, dstr)
    hs2 = _combine(y1, bmat, xp, degr[0])
    y2 = _spmv_call(hs2, srcr, dstr)
    out = _mlp(y2, bmat, xp, W1.astype(F32), b1.reshape(1, D_OUT),
               gamma.reshape(1, D_OUT), beta.reshape(1, D_OUT),
               W2.astype(F32), b2.reshape(1, D_OUT))
    return out[:N]


# speedup vs baseline: 9.1601x; 1.2316x over previous
import jax
import jax.numpy as jnp
from jax import lax
from jax.experimental import pallas as pl
from jax.experimental.pallas import tpu as pltpu
from jax.experimental.pallas import tpu_sc as plsc

N = 10000
NP = 10240
E = 320000
NT = 32
CH = 128
NCH = 79
EPT = NCH * CH
EP = EPT * NT
DEG_NCH = 157
EPD = DEG_NCH * CH * 16
D_IN = 128
D_OUT = 512
RB = 1280
F32 = jnp.float32


def _mesh():
    return plsc.VectorSubcoreMesh(core_axis_name="c", subcore_axis_name="s")


def _deg_body(cidx_hbm, deg_hbm, acc, cidx_v, onesb, semS):
    c = lax.axis_index("c")
    s = lax.axis_index("s")
    zero16 = jnp.zeros((16,), F32)
    one16 = jnp.full((16,), 1.0, F32)

    def zrow(i, _):
        for jj in range(8):
            onesb[i, pl.ds(jj * 16, 16)] = zero16
        return 0

    lax.fori_loop(0, 128, zrow, 0)

    def zero_acc(k, _):
        pltpu.sync_copy(onesb, acc.at[pl.ds(s * 640 + k * 128, 128)])
        return 0

    lax.fori_loop(0, 5, zero_acc, 0)

    def orow(i, _):
        for jj in range(8):
            onesb[i, pl.ds(jj * 16, 16)] = one16
        return 0

    lax.fori_loop(0, 128, orow, 0)
    plsc.subcore_barrier()

    pltpu.sync_copy(cidx_hbm.at[c, s], cidx_v)

    def fire(j, _):
        pltpu.async_copy(onesb, acc.at[cidx_v.at[j]], semS, add=True)
        return 0

    lax.fori_loop(0, 8, fire, 0)

    def add_chunk(j, _):
        pltpu.make_async_copy(onesb, acc.at[cidx_v.at[0]], semS).wait()
        pltpu.async_copy(onesb, acc.at[cidx_v.at[j + 8]], semS, add=True)
        return 0

    lax.fori_loop(0, DEG_NCH - 8, add_chunk, 0)

    def drain(j, _):
        pltpu.make_async_copy(onesb, acc.at[cidx_v.at[0]], semS).wait()
        return 0

    lax.fori_loop(0, 8, drain, 0)
    plsc.subcore_barrier()

    pltpu.sync_copy(acc.at[pl.ds(s * 640, 640)],
                    deg_hbm.at[c, pl.ds(s * 640, 640)])


def _deg_call(cidx):
    fn = pl.kernel(
        _deg_body,
        out_type=jax.ShapeDtypeStruct((2, NP, 128), F32),
        mesh=_mesh(),
        scratch_types=[
            pltpu.VMEM_SHARED((NP, 128), F32),
            pltpu.VMEM((DEG_NCH, 128), jnp.int32),
            pltpu.VMEM((128, 128), F32),
            pltpu.SemaphoreType.DMA,
        ],
    )
    return fn(cidx)


def _spmv_body(hs_hbm, eidx_hbm, y_hbm, acc, rows, e0, e1, e2, e3,
               semg, semi):
    c = lax.axis_index("c")
    s = lax.axis_index("s")
    wid = c * 16 + s
    eslot = [e0, e1, e2, e3]

    zero16 = jnp.zeros((16,), F32)

    def init_row(i, _):
        for jj in range(8):
            rows[0, i, pl.ds(jj * 16, 16)] = zero16
        return 0

    lax.fori_loop(0, CH, init_row, 0)

    def zero_acc(k, _):
        pltpu.sync_copy(rows.at[0], acc.at[pl.ds(s * 640 + k * 128, 128)])
        return 0

    lax.fori_loop(0, 5, zero_acc, 0)
    plsc.subcore_barrier()

    pltpu.sync_copy(eidx_hbm.at[wid, 0], e0)
    pltpu.async_copy(hs_hbm.at[e0.at[0]], rows.at[0], semg.at[0])
    for k in range(1, 4):
        pltpu.async_copy(eidx_hbm.at[wid, k], eslot[k], semi.at[k])

    def chunk_ops(j, k):
        kn = (k + 1) % 4
        pltpu.make_async_copy(eidx_hbm.at[wid, 0], eslot[kn],
                              semi.at[kn]).wait()
        pltpu.async_copy(hs_hbm.at[eslot[kn].at[0]], rows.at[(k + 1) % 2],
                         semg.at[(k + 1) % 2])
        pltpu.make_async_copy(hs_hbm.at[eslot[k].at[0]], rows.at[k % 2],
                              semg.at[k % 2]).wait()
        pltpu.sync_copy(rows.at[k % 2], acc.at[eslot[k].at[1]], add=True)
        pltpu.async_copy(eidx_hbm.at[wid, j + 4], eslot[k], semi.at[k])

    def quad(q, _):
        for k in range(4):
            chunk_ops(4 * q + k, k)
        return 0

    lax.fori_loop(0, NCH // 4, quad, 0)
    for j in range(NCH - NCH % 4, NCH):
        chunk_ops(j, j % 4)

    pltpu.make_async_copy(hs_hbm.at[e0.at[0]], rows.at[NCH % 2],
                          semg.at[NCH % 2]).wait()
    for j in range(NCH + 1, NCH + 4):
        pltpu.make_async_copy(eidx_hbm.at[wid, 0], eslot[j % 4],
                              semi.at[j % 4]).wait()
    plsc.subcore_barrier()

    pltpu.sync_copy(acc.at[pl.ds(s * 640, 640)],
                    y_hbm.at[c, pl.ds(s * 640, 640)])


def _spmv_call(hs, eidx):
    fn = pl.kernel(
        _spmv_body,
        out_type=jax.ShapeDtypeStruct((2, NP, D_IN), F32),
        mesh=_mesh(),
        scratch_types=[
            pltpu.VMEM_SHARED((NP, D_IN), F32),
            pltpu.VMEM((2, CH, D_IN), F32),
            pltpu.VMEM((2, CH), jnp.int32),
            pltpu.VMEM((2, CH), jnp.int32),
            pltpu.VMEM((2, CH), jnp.int32),
            pltpu.VMEM((2, CH), jnp.int32),
            pltpu.SemaphoreType.DMA((2,)),
            pltpu.SemaphoreType.DMA((4,)),
        ],
    )
    return fn(hs, eidx)


def _scale_in_body(dsrc, ddst, x, hs, bmat):
    a = lax.rsqrt(jnp.maximum(dsrc[...], 1.0))
    hs[...] = x[...] * a
    bmat[...] = lax.rsqrt(jnp.maximum(ddst[...], 1.0))


def _scale_in(dsrc, ddst, xp):
    grid = NP // RB
    return pl.pallas_call(
        _scale_in_body,
        grid=(grid,),
        in_specs=[
            pl.BlockSpec((RB, D_IN), lambda r: (r, 0)),
            pl.BlockSpec((RB, D_IN), lambda r: (r, 0)),
            pl.BlockSpec((RB, D_IN), lambda r: (r, 0)),
        ],
        out_specs=[
            pl.BlockSpec((RB, D_IN), lambda r: (r, 0)),
            pl.BlockSpec((RB, D_IN), lambda r: (r, 0)),
        ],
        out_shape=[
            jax.ShapeDtypeStruct((NP, D_IN), F32),
            jax.ShapeDtypeStruct((NP, D_IN), F32),
        ],
    )(dsrc, ddst, xp)


def _combine_body(y, bmat, x, dsrc, hs2):
    h1 = (y[0] + y[1]) * bmat[...] + x[...]
    a = lax.rsqrt(jnp.maximum(dsrc[...], 1.0))
    hs2[...] = h1 * a


def _combine(y1, bmat, xp, dsrc):
    grid = NP // RB
    return pl.pallas_call(
        _combine_body,
        grid=(grid,),
        in_specs=[
            pl.BlockSpec((2, RB, D_IN), lambda r: (0, r, 0)),
            pl.BlockSpec((RB, D_IN), lambda r: (r, 0)),
            pl.BlockSpec((RB, D_IN), lambda r: (r, 0)),
            pl.BlockSpec((RB, D_IN), lambda r: (r, 0)),
        ],
        out_specs=pl.BlockSpec((RB, D_IN), lambda r: (r, 0)),
        out_shape=jax.ShapeDtypeStruct((NP, D_IN), F32),
    )(y1, bmat, xp, dsrc)


def _mlp_body(y, bmat, x, w1, b1, g, bt, w2, b2, out):
    h2 = (y[0] + y[1]) * bmat[...] + x[...]
    e = jnp.dot(h2, w1[...], preferred_element_type=F32,
                precision=lax.Precision.HIGHEST) + b1[...]
    e = jnp.maximum(e, 0.0)
    mu = jnp.mean(e, axis=-1, keepdims=True)
    var = jnp.mean((e - mu) ** 2, axis=-1, keepdims=True)
    e = g[...] * (e - mu) * lax.rsqrt(var + 1e-5) + bt[...]
    o = jnp.dot(e, w2[...], preferred_element_type=F32,
                precision=lax.Precision.HIGHEST) + b2[...]
    nrm = jnp.sqrt(jnp.sum(o * o, axis=-1, keepdims=True))
    out[...] = o / jnp.maximum(nrm, 1e-8)


def _mlp(y2, bmat, xp, W1, b1, gamma, beta, W2, b2):
    grid = NP // RB
    return pl.pallas_call(
        _mlp_body,
        grid=(grid,),
        in_specs=[
            pl.BlockSpec((2, RB, D_IN), lambda r: (0, r, 0)),
            pl.BlockSpec((RB, D_IN), lambda r: (r, 0)),
            pl.BlockSpec((RB, D_IN), lambda r: (r, 0)),
            pl.BlockSpec((D_IN, D_OUT), lambda r: (0, 0)),
            pl.BlockSpec((1, D_OUT), lambda r: (0, 0)),
            pl.BlockSpec((1, D_OUT), lambda r: (0, 0)),
            pl.BlockSpec((1, D_OUT), lambda r: (0, 0)),
            pl.BlockSpec((D_OUT, D_OUT), lambda r: (0, 0)),
            pl.BlockSpec((1, D_OUT), lambda r: (0, 0)),
        ],
        out_specs=pl.BlockSpec((RB, D_OUT), lambda r: (r, 0)),
        out_shape=jax.ShapeDtypeStruct((NP, D_OUT), F32),
    )(y2, bmat, xp, W1, b1, gamma, beta, W2, b2)


def kernel(x, edge_index, W1, b1, gamma, beta, W2, b2):
    src = edge_index[0]
    dst = edge_index[1]
    pad_spmv = jnp.full((EP - E,), N, dtype=jnp.int32)
    src_p = jnp.concatenate([src, pad_spmv])
    dst_p = jnp.concatenate([dst, pad_spmv])
    srcr = src_p.reshape(NT, NCH, CH)
    dstr = dst_p.reshape(NT, NCH, CH)
    eidx = jnp.stack([srcr, dstr], axis=2)
    eidx = jnp.concatenate([eidx, eidx[:, 0:4]], axis=1)
    pad_deg = jnp.full((EPD - E,), N, dtype=jnp.int32)
    cidx = jnp.stack([
        jnp.concatenate([src, pad_deg]),
        jnp.concatenate([dst, pad_deg]),
    ]).reshape(2, 16, DEG_NCH, CH)
    xp = jnp.pad(x, ((0, NP - N), (0, 0)))

    degr = _deg_call(cidx)
    hs1, bmat = _scale_in(degr[0], degr[1], xp)
    y1 = _spmv_call(hs1, eidx)
    hs2 = _combine(y1, bmat, xp, degr[0])
    y2 = _spmv_call(hs2, eidx)
    out = _mlp(y2, bmat, xp, W1.astype(F32), b1.reshape(1, D_OUT),
               gamma.reshape(1, D_OUT), beta.reshape(1, D_OUT),
               W2.astype(F32), b2.reshape(1, D_OUT))
    return out[:N]
